# baseline (speedup 1.0000x reference)
import functools

import jax
import jax.numpy as jnp
from jax.experimental import pallas as pl

THETA = 10000.0
NEG_INF = -1e9


def _rope_tables(pid, blk, half):
    pos = (pid * blk
           + jax.lax.broadcasted_iota(jnp.int32, (blk, 1), 0)).astype(
               jnp.float32)
    idx = jax.lax.broadcasted_iota(jnp.int32, (1, half), 1).astype(jnp.float32)
    freqs = jnp.exp(idx * (-jnp.log(THETA) / half))
    ang = pos * freqs
    return jnp.cos(ang), jnp.sin(ang)


def _prologue_kernel(x_ref, wqn_ref, wqr_ref, wdkv_ref, wkr_ref, wuk_ref,
                     wuv_ref, qn_ref, qr_ref, kn_ref, kr_ref, v_ref,
                     *, n_head, d_r):
    blk = x_ref.shape[0]
    xb = x_ref[...].astype(jnp.bfloat16)
    f32 = jnp.float32
    qn = jnp.dot(xb, wqn_ref[...], preferred_element_type=f32)
    qr = jnp.dot(xb, wqr_ref[...], preferred_element_type=f32)
    c = jnp.dot(xb, wdkv_ref[...], preferred_element_type=f32)
    krb = jnp.dot(xb, wkr_ref[...], preferred_element_type=f32)
    c16 = c.astype(jnp.bfloat16)
    kn = jnp.dot(c16, wuk_ref[...], preferred_element_type=f32)
    v = jnp.dot(c16, wuv_ref[...], preferred_element_type=f32)

    half = d_r // 2
    cos, sin = _rope_tables(pl.program_id(0), blk, half)
    q3 = qr.reshape(blk, n_head, d_r)
    q1, q2 = q3[..., :half], q3[..., half:]
    c3, s3 = cos[:, None, :], sin[:, None, :]
    qr_out = jnp.concatenate([q1 * c3 - q2 * s3, q1 * s3 + q2 * c3], axis=-1)
    k1, k2 = krb[:, :half], krb[:, half:]
    kr_out = jnp.concatenate([k1 * cos - k2 * sin, k1 * sin + k2 * cos],
                             axis=-1)

    d_h = qn.shape[1] // n_head
    qn_ref[...] = qn.reshape(blk, n_head, d_h).swapaxes(0, 1).astype(
        jnp.bfloat16)
    qr_ref[...] = qr_out.swapaxes(0, 1).astype(jnp.bfloat16)
    kn_ref[...] = kn.reshape(blk, n_head, d_h).swapaxes(0, 1).astype(
        jnp.bfloat16)
    kr_ref[...] = kr_out.astype(jnp.bfloat16)
    v_ref[...] = v.reshape(blk, n_head, d_h).swapaxes(0, 1).astype(
        jnp.bfloat16)


def _flash_kernel(qn_ref, qr_ref, kn_ref, kr_ref, v_ref, o_ref,
                  *, q_blk, k_blk, scale):
    qi = pl.program_id(1)
    qn = qn_ref[0]
    qr = qr_ref[0]
    d_h = qn.shape[1]
    f32 = jnp.float32
    row = qi * q_blk + jax.lax.broadcasted_iota(jnp.int32, (q_blk, k_blk), 0)
    col_base = jax.lax.broadcasted_iota(jnp.int32, (q_blk, k_blk), 1)

    n_steps = (qi * q_blk) // k_blk + 1
    m0 = jnp.full((q_blk, 1), NEG_INF, f32)
    l0 = jnp.zeros((q_blk, 1), f32)
    acc0 = jnp.zeros((q_blk, d_h), f32)

    def step(j, carry):
        m, l, acc = carry
        kb = kn_ref[0, pl.ds(j * k_blk, k_blk), :]
        krb = kr_ref[pl.ds(j * k_blk, k_blk), :]
        s = jax.lax.dot_general(qn, kb, (((1,), (1,)), ((), ())),
                                preferred_element_type=f32)
        s += jax.lax.dot_general(qr, krb, (((1,), (1,)), ((), ())),
                                 preferred_element_type=f32)
        s *= scale
        s = jnp.where(j * k_blk + col_base <= row, s, NEG_INF)
        cur_max = s.max(axis=-1, keepdims=True)
        m_new = jnp.maximum(m, cur_max)
        p = jnp.exp(s - m_new)
        alpha = jnp.exp(m - m_new)
        l_new = l * alpha + p.sum(axis=-1, keepdims=True)
        vb = v_ref[0, pl.ds(j * k_blk, k_blk), :]
        acc_new = acc * alpha + jax.lax.dot_general(
            p.astype(jnp.bfloat16), vb, (((1,), (0,)), ((), ())),
            preferred_element_type=f32)
        return m_new, l_new, acc_new

    m, l, acc = jax.lax.fori_loop(0, n_steps, step, (m0, l0, acc0))
    o_ref[0] = (acc / l).astype(jnp.bfloat16)


def _post_kernel(o_ref, wo_ref, x_ref, g1_ref, b1_ref, wg_ref,
                 x1_ref, gates_ref, fi_ref, pi_ref, lose_ref,
                 *, n_exp, n_blocks):
    pid = pl.program_id(0)
    f32 = jnp.float32
    n_head, blk2, d_h = o_ref.shape
    o = o_ref[...].swapaxes(0, 1).reshape(blk2, n_head * d_h)
    att = jnp.dot(o, wo_ref[...], preferred_element_type=f32)
    y = x_ref[...] + att
    mu = y.mean(axis=-1, keepdims=True)
    var = ((y - mu) ** 2).mean(axis=-1, keepdims=True)
    x1 = (y - mu) / jnp.sqrt(var + 1e-5) * g1_ref[...] + b1_ref[...]
    x1_ref[...] = x1

    logits = jnp.dot(x1, wg_ref[...], preferred_element_type=f32,
                     precision=jax.lax.Precision.HIGHEST)
    mx = logits.max(axis=-1, keepdims=True)
    ex = jnp.exp(logits - mx)
    probs = ex / ex.sum(axis=-1, keepdims=True)

    blk = probs.shape[0]
    e_iota = jax.lax.broadcasted_iota(jnp.int32, (blk, n_exp), 1)
    m1 = probs.max(axis=-1, keepdims=True)
    i1 = jnp.where(probs == m1, e_iota, n_exp).min(axis=-1, keepdims=True)
    oh1 = e_iota == i1
    masked = jnp.where(oh1, -1.0, probs)
    m2 = masked.max(axis=-1, keepdims=True)
    i2 = jnp.where(masked == m2, e_iota, n_exp).min(axis=-1, keepdims=True)
    oh2 = e_iota == i2
    denom = m1 + m2
    gates = jnp.where(oh1, m1 / denom, 0.0) + jnp.where(oh2, m2 / denom, 0.0)
    gates_ref[...] = gates

    fi_part = (oh1.astype(f32) + oh2.astype(f32)).sum(axis=0, keepdims=True)
    pi_part = probs.sum(axis=0, keepdims=True)

    @pl.when(pid == 0)
    def _():
        fi_ref[...] = jnp.zeros_like(fi_ref)
        pi_ref[...] = jnp.zeros_like(pi_ref)

    fi_ref[...] += fi_part
    pi_ref[...] += pi_part

    @pl.when(pid == n_blocks - 1)
    def _():
        total = jnp.float32(blk * n_blocks)
        val = n_exp * (fi_ref[...] * pi_ref[...]).sum() / (total * total)
        lose_ref[...] = jnp.reshape(val, (1, 1))


def _moe_kernel(x1_ref, gates_ref, w1_ref, w2_ref, g2_ref, b2_ref, out_ref,
                *, n_exp):
    e = pl.program_id(1)
    f32 = jnp.float32
    gates = gates_ref[...]
    e_iota = jax.lax.broadcasted_iota(jnp.int32, gates.shape, 1)
    gsel = jnp.where(e_iota == e, gates, 0.0).sum(axis=-1, keepdims=True)
    g = gsel + (e == n_exp).astype(f32)
    x1b = x1_ref[...].astype(jnp.bfloat16)
    hpre = jnp.dot(x1b, w1_ref[0], preferred_element_type=f32)
    h = (hpre * jax.nn.sigmoid(hpre)).astype(jnp.bfloat16)
    eo = jnp.dot(h, w2_ref[0], preferred_element_type=f32)
    contrib = g * eo

    @pl.when(e == 0)
    def _():
        out_ref[...] = contrib

    @pl.when(e > 0)
    def _():
        out_ref[...] += contrib

    @pl.when(e == n_exp)
    def _():
        y = x1_ref[...] + out_ref[...]
        mu = y.mean(axis=-1, keepdims=True)
        var = ((y - mu) ** 2).mean(axis=-1, keepdims=True)
        out_ref[...] = (y - mu) / jnp.sqrt(var + 1e-5) * g2_ref[...] + b2_ref[...]


def kernel(x, Wq_nope, Wq_rope, W_dkv, W_kr, W_uk, W_uv, W_o, ln1_g, ln1_b,
           ln2_g, ln2_b, W_gate, We1, We2, Ws1, Ws2):
    b, s, d = x.shape
    d_c = W_dkv.shape[1]
    d_r = W_kr.shape[1]
    n_head = Wq_rope.shape[1] // d_r
    d_h = Wq_nope.shape[1] // n_head
    n_exp = W_gate.shape[1]
    hidden = We1.shape[2]
    xs = x.reshape(s, d)

    blk = min(256, s)
    n_blocks = s // blk
    bf16 = jnp.bfloat16
    f32 = jnp.float32

    qn, qr, kn, kr, v = pl.pallas_call(
        functools.partial(_prologue_kernel, n_head=n_head, d_r=d_r),
        grid=(n_blocks,),
        in_specs=[
            pl.BlockSpec((blk, d), lambda i: (i, 0)),
            pl.BlockSpec((d, n_head * d_h), lambda i: (0, 0)),
            pl.BlockSpec((d, n_head * d_r), lambda i: (0, 0)),
            pl.BlockSpec((d, d_c), lambda i: (0, 0)),
            pl.BlockSpec((d, d_r), lambda i: (0, 0)),
            pl.BlockSpec((d_c, n_head * d_h), lambda i: (0, 0)),
            pl.BlockSpec((d_c, n_head * d_h), lambda i: (0, 0)),
        ],
        out_specs=[
            pl.BlockSpec((n_head, blk, d_h), lambda i: (0, i, 0)),
            pl.BlockSpec((n_head, blk, d_r), lambda i: (0, i, 0)),
            pl.BlockSpec((n_head, blk, d_h), lambda i: (0, i, 0)),
            pl.BlockSpec((blk, d_r), lambda i: (i, 0)),
            pl.BlockSpec((n_head, blk, d_h), lambda i: (0, i, 0)),
        ],
        out_shape=[
            jax.ShapeDtypeStruct((n_head, s, d_h), bf16),
            jax.ShapeDtypeStruct((n_head, s, d_r), bf16),
            jax.ShapeDtypeStruct((n_head, s, d_h), bf16),
            jax.ShapeDtypeStruct((s, d_r), bf16),
            jax.ShapeDtypeStruct((n_head, s, d_h), bf16),
        ],
    )(xs, Wq_nope.astype(bf16), Wq_rope.astype(bf16), W_dkv.astype(bf16),
      W_kr.astype(bf16), W_uk.astype(bf16), W_uv.astype(bf16))

    q_blk = blk
    k_blk = blk
    scale = 1.0 / (d_h + d_r) ** 0.5
    o = pl.pallas_call(
        functools.partial(_flash_kernel, q_blk=q_blk, k_blk=k_blk,
                          scale=scale),
        grid=(n_head, n_blocks),
        in_specs=[
            pl.BlockSpec((1, q_blk, d_h), lambda h, i: (h, i, 0)),
            pl.BlockSpec((1, q_blk, d_r), lambda h, i: (h, i, 0)),
            pl.BlockSpec((1, s, d_h), lambda h, i: (h, 0, 0)),
            pl.BlockSpec((s, d_r), lambda h, i: (0, 0)),
            pl.BlockSpec((1, s, d_h), lambda h, i: (h, 0, 0)),
        ],
        out_specs=pl.BlockSpec((1, q_blk, d_h), lambda h, i: (h, i, 0)),
        out_shape=jax.ShapeDtypeStruct((n_head, s, d_h), bf16),
    )(qn, qr, kn, kr, v)

    x1, gates, fi, pi, lose = pl.pallas_call(
        functools.partial(_post_kernel, n_exp=n_exp, n_blocks=n_blocks),
        grid=(n_blocks,),
        in_specs=[
            pl.BlockSpec((n_head, blk, d_h), lambda i: (0, i, 0)),
            pl.BlockSpec((n_head * d_h, d), lambda i: (0, 0)),
            pl.BlockSpec((blk, d), lambda i: (i, 0)),
            pl.BlockSpec((1, d), lambda i: (0, 0)),
            pl.BlockSpec((1, d), lambda i: (0, 0)),
            pl.BlockSpec((d, n_exp), lambda i: (0, 0)),
        ],
        out_specs=[
            pl.BlockSpec((blk, d), lambda i: (i, 0)),
            pl.BlockSpec((blk, n_exp), lambda i: (i, 0)),
            pl.BlockSpec((1, n_exp), lambda i: (0, 0)),
            pl.BlockSpec((1, n_exp), lambda i: (0, 0)),
            pl.BlockSpec((1, 1), lambda i: (0, 0)),
        ],
        out_shape=[
            jax.ShapeDtypeStruct((s, d), f32),
            jax.ShapeDtypeStruct((s, n_exp), f32),
            jax.ShapeDtypeStruct((1, n_exp), f32),
            jax.ShapeDtypeStruct((1, n_exp), f32),
            jax.ShapeDtypeStruct((1, 1), f32),
        ],
    )(o, W_o.astype(bf16), xs, ln1_g.reshape(1, d), ln1_b.reshape(1, d),
      W_gate)

    w1 = jnp.concatenate([We1, Ws1], axis=0).astype(bf16)
    w2 = jnp.concatenate([We2, Ws2], axis=0).astype(bf16)
    x2 = pl.pallas_call(
        functools.partial(_moe_kernel, n_exp=n_exp),
        grid=(n_blocks, n_exp + 1),
        in_specs=[
            pl.BlockSpec((blk, d), lambda i, e: (i, 0)),
            pl.BlockSpec((blk, n_exp), lambda i, e: (i, 0)),
            pl.BlockSpec((1, d, hidden), lambda i, e: (e, 0, 0)),
            pl.BlockSpec((1, hidden, d), lambda i, e: (e, 0, 0)),
            pl.BlockSpec((1, d), lambda i, e: (0, 0)),
            pl.BlockSpec((1, d), lambda i, e: (0, 0)),
        ],
        out_specs=pl.BlockSpec((blk, d), lambda i, e: (i, 0)),
        out_shape=jax.ShapeDtypeStruct((s, d), f32),
    )(x1, gates, w1, w2, ln2_g.reshape(1, d), ln2_b.reshape(1, d))

    return x2.reshape(b, s, d), lose.reshape(())
